# baseline (speedup 1.0000x reference)
import functools

import jax
import jax.numpy as jnp
from jax import lax
from jax.experimental import pallas as pl
from jax.experimental.pallas import tpu as pltpu
from jax.experimental.pallas import tpu_sc as plsc

NC = 2
NS = 16
NW = NC * NS
L = 16


def _sc_segment_matvec(n_cats_pad, n_pix, feat, e_total, chunk):
    per_w = e_total // NW
    n_chunks = per_w // chunk
    rows_per_tile = n_cats_pad // NS
    assert n_cats_pad % (NS * 8) == 0 and per_w % chunk == 0
    assert rows_per_tile % chunk == 0 and chunk % L == 0

    mesh = plsc.VectorSubcoreMesh(core_axis_name="c", subcore_axis_name="s")

    @functools.partial(
        pl.kernel,
        out_type=jax.ShapeDtypeStruct((NC, n_cats_pad, feat), jnp.float32),
        mesh=mesh,
        scratch_types=[
            pltpu.VMEM_SHARED((n_pix,), jnp.int32),
            pltpu.VMEM_SHARED((n_cats_pad, feat), jnp.float32),
            pltpu.VMEM((6, 1, chunk), jnp.int32),
            pltpu.VMEM((4, chunk), jnp.int32),
            pltpu.VMEM((4, chunk), jnp.float32),
            pltpu.VMEM((3, chunk), jnp.int32),
            pltpu.VMEM((3, chunk, feat), jnp.float32),
            pltpu.SemaphoreType.DMA((4,)),
            pltpu.SemaphoreType.DMA((4,)),
            pltpu.SemaphoreType.DMA((6,)),
            pltpu.SemaphoreType.DMA((3,)),
            pltpu.SemaphoreType.DMA((3,)),
            pltpu.SemaphoreType.DMA((2,)),
        ],
        compiler_params=pltpu.CompilerParams(needs_layout_passes=False),
    )
    def sc_kernel(runoff_hbm, pix_hbm, src_hbm, dest_hbm, w_hbm, out_hbm,
                  pix_tab, acc, dvb, srcb, wb, gv, rows,
                  sem_s, sem_w, sem_d, sem_g, sem_r, sem_a):
        cid = lax.axis_index("c")
        sid = lax.axis_index("s")
        wid = sid * NC + cid

        def _zero_body(e, _):
            for f in range(feat // L):
                rows[0, e, pl.ds(f * L, L)] = jnp.zeros((L,), jnp.float32)
            return 0
        lax.fori_loop(0, chunk, _zero_body, 0)
        for k in range(rows_per_tile // chunk):
            pltpu.sync_copy(rows.at[0],
                            acc.at[pl.ds(sid * rows_per_tile + k * chunk,
                                         chunk)])

        @pl.when(sid == 0)
        def _():
            pltpu.sync_copy(pix_hbm, pix_tab)
        plsc.subcore_barrier()

        def _issue_swd(j):
            base = wid * per_w + j * chunk
            s = j % 4
            pltpu.async_copy(src_hbm.at[pl.ds(base, chunk)], srcb.at[s],
                             sem_s.at[s])
            pltpu.async_copy(w_hbm.at[pl.ds(base, chunk)], wb.at[s],
                             sem_w.at[s])
            pltpu.async_copy(dest_hbm.at[wid, j], dvb.at[j % 6],
                             sem_d.at[j % 6])

        def _wait_swd(j):
            s = j % 4
            base = wid * per_w + j * chunk
            pltpu.make_async_copy(src_hbm.at[pl.ds(base, chunk)], srcb.at[s],
                                  sem_s.at[s]).wait()
            pltpu.make_async_copy(w_hbm.at[pl.ds(base, chunk)], wb.at[s],
                                  sem_w.at[s]).wait()
            pltpu.make_async_copy(dest_hbm.at[wid, j], dvb.at[j % 6],
                                  sem_d.at[j % 6]).wait()

        def _issue_xlate(j):
            pltpu.async_copy(pix_tab.at[srcb.at[j % 4]], gv.at[j % 3],
                             sem_g.at[j % 3])

        def _wait_xlate(j):
            pltpu.make_async_copy(pix_tab.at[srcb.at[j % 4]], gv.at[j % 3],
                                  sem_g.at[j % 3]).wait()

        def _issue_rows(j):
            pltpu.async_copy(runoff_hbm.at[gv.at[j % 3]], rows.at[j % 3],
                             sem_r.at[j % 3])

        def _wait_rows(j):
            pltpu.make_async_copy(runoff_hbm.at[gv.at[j % 3]],
                                  rows.at[j % 3], sem_r.at[j % 3]).wait()

        def _issue_scatter(j):
            pltpu.async_copy(rows.at[j % 3], acc.at[dvb.at[j % 6, 0]],
                             sem_a.at[j % 2], add=True)

        def _wait_scatter(j):
            pltpu.make_async_copy(rows.at[j % 3], acc.at[dvb.at[j % 6, 0]],
                                  sem_a.at[j % 2]).wait()

        _issue_swd(0)
        _issue_swd(1)
        _issue_swd(2)
        _wait_swd(0)
        _issue_xlate(0)
        _wait_xlate(0)
        _issue_rows(0)
        _wait_swd(1)
        _issue_xlate(1)

        def _chunk_body(j, _):
            @pl.when(j + 3 < n_chunks)
            def _():
                _issue_swd(j + 3)

            @pl.when(j + 2 < n_chunks)
            def _():
                _wait_swd(j + 2)
                _issue_xlate(j + 2)

            @pl.when(j + 1 < n_chunks)
            def _():
                _wait_xlate(j + 1)

                @pl.when(j >= 2)
                def _():
                    _wait_scatter(j - 2)
                _issue_rows(j + 1)

            _wait_rows(j)
            p = j % 3
            def _scale(i, _):
                base16 = i * L
                for r in range(L):
                    e = base16 + r
                    wbc = plsc.load_gather(
                        wb.at[j % 4], [jnp.full((L,), e, jnp.int32)])
                    for f in range(feat // L):
                        sl = pl.ds(f * L, L)
                        rows[p, e, sl] = rows[p, e, sl] * wbc
                return 0
            lax.fori_loop(0, chunk // L, _scale, 0)
            _issue_scatter(j)
            return 0

        lax.fori_loop(0, n_chunks, _chunk_body, 0)
        _wait_scatter(n_chunks - 2)
        _wait_scatter(n_chunks - 1)
        plsc.subcore_barrier()

        for k in range(rows_per_tile // chunk):
            off = sid * rows_per_tile + k * chunk
            pltpu.sync_copy(acc.at[pl.ds(off, chunk)],
                            out_hbm.at[cid, pl.ds(off, chunk)])

    return sc_kernel


def _tc_add(a, b):
    n, f = a.shape
    blk = 1024

    def body(a_ref, b_ref, o_ref):
        o_ref[...] = a_ref[...] + b_ref[...]

    return pl.pallas_call(
        body,
        grid=(n // blk,),
        in_specs=[pl.BlockSpec((blk, f), lambda i: (i, 0))] * 2,
        out_specs=pl.BlockSpec((blk, f), lambda i: (i, 0)),
        out_shape=jax.ShapeDtypeStruct((n, f), jnp.float32),
    )(a, b)


def kernel(runoff, pix_idxs, src_idxs, dest_idxs, weights):
    n_pix = pix_idxs.shape[0]
    e_total = src_idxs.shape[0]
    n_cats = 10000
    n_cats_pad = 10240
    feat = runoff.shape[1]
    chunk = 80

    pix32 = pix_idxs.astype(jnp.int32)
    src32 = src_idxs.astype(jnp.int32)
    per_w = e_total // NW
    dest32 = dest_idxs.astype(jnp.int32).reshape(NW, per_w // chunk, 1, chunk)

    sc = _sc_segment_matvec(n_cats_pad, n_pix, feat, e_total, chunk)
    partials = sc(runoff, pix32, src32, dest32, weights)
    out = _tc_add(partials[0], partials[1])
    return out[:n_cats][None]


# speedup vs baseline: 2.3297x; 2.3297x over previous
import functools

import jax
import jax.numpy as jnp
from jax import lax
from jax.experimental import pallas as pl
from jax.experimental.pallas import tpu as pltpu
from jax.experimental.pallas import tpu_sc as plsc

NC = 2
NS = 16
NW = NC * NS
L = 16


def _sc_segment_matvec(n_cats_pad, n_pix, feat, e_total, chunk):
    per_w = e_total // NW
    n_chunks = per_w // chunk
    rows_per_tile = n_cats_pad // NS
    assert n_cats_pad % (NS * 8) == 0 and per_w % chunk == 0
    assert rows_per_tile % chunk == 0 and chunk % L == 0

    mesh = plsc.VectorSubcoreMesh(core_axis_name="c", subcore_axis_name="s")

    @functools.partial(
        pl.kernel,
        out_type=jax.ShapeDtypeStruct((NC, n_cats_pad, feat), jnp.float32),
        mesh=mesh,
        scratch_types=[
            pltpu.VMEM_SHARED((n_pix,), jnp.int32),
            pltpu.VMEM_SHARED((n_cats_pad, feat), jnp.float32),
            pltpu.VMEM((6, 1, chunk), jnp.int32),
            pltpu.VMEM((4, chunk), jnp.int32),
            pltpu.VMEM((4, chunk), jnp.float32),
            pltpu.VMEM((3, chunk), jnp.int32),
            pltpu.VMEM((3, chunk, feat), jnp.float32),
            pltpu.SemaphoreType.DMA((4,)),
            pltpu.SemaphoreType.DMA((4,)),
            pltpu.SemaphoreType.DMA((6,)),
            pltpu.SemaphoreType.DMA((3,)),
            pltpu.SemaphoreType.DMA((3,)),
            pltpu.SemaphoreType.DMA((2,)),
        ],
        compiler_params=pltpu.CompilerParams(needs_layout_passes=False),
    )
    def sc_kernel(runoff_hbm, pix_hbm, src_hbm, dest_hbm, w_hbm, out_hbm,
                  pix_tab, acc, dvb, srcb, wb, gv, rows,
                  sem_s, sem_w, sem_d, sem_g, sem_r, sem_a):
        cid = lax.axis_index("c")
        sid = lax.axis_index("s")
        wid = sid * NC + cid

        def _zero_body(e, _):
            for f in range(feat // L):
                rows[0, e, pl.ds(f * L, L)] = jnp.zeros((L,), jnp.float32)
            return 0
        lax.fori_loop(0, chunk, _zero_body, 0)
        for k in range(rows_per_tile // chunk):
            pltpu.sync_copy(rows.at[0],
                            acc.at[pl.ds(sid * rows_per_tile + k * chunk,
                                         chunk)])

        @pl.when(sid == 0)
        def _():
            pltpu.sync_copy(pix_hbm, pix_tab)
        plsc.subcore_barrier()

        def _issue_swd(j):
            base = wid * per_w + j * chunk
            s = j % 4
            pltpu.async_copy(src_hbm.at[pl.ds(base, chunk)], srcb.at[s],
                             sem_s.at[s])
            pltpu.async_copy(w_hbm.at[pl.ds(base, chunk)], wb.at[s],
                             sem_w.at[s])
            pltpu.async_copy(dest_hbm.at[wid, j], dvb.at[j % 6],
                             sem_d.at[j % 6])

        def _wait_swd(j):
            s = j % 4
            base = wid * per_w + j * chunk
            pltpu.make_async_copy(src_hbm.at[pl.ds(base, chunk)], srcb.at[s],
                                  sem_s.at[s]).wait()
            pltpu.make_async_copy(w_hbm.at[pl.ds(base, chunk)], wb.at[s],
                                  sem_w.at[s]).wait()
            pltpu.make_async_copy(dest_hbm.at[wid, j], dvb.at[j % 6],
                                  sem_d.at[j % 6]).wait()

        def _issue_xlate(j):
            pltpu.async_copy(pix_tab.at[srcb.at[j % 4]], gv.at[j % 3],
                             sem_g.at[j % 3])

        def _wait_xlate(j):
            pltpu.make_async_copy(pix_tab.at[srcb.at[j % 4]], gv.at[j % 3],
                                  sem_g.at[j % 3]).wait()

        def _issue_rows(j):
            pltpu.async_copy(runoff_hbm.at[gv.at[j % 3]], rows.at[j % 3],
                             sem_r.at[j % 3])

        def _wait_rows(j):
            pltpu.make_async_copy(runoff_hbm.at[gv.at[j % 3]],
                                  rows.at[j % 3], sem_r.at[j % 3]).wait()

        def _issue_scatter(j):
            pltpu.async_copy(rows.at[j % 3], acc.at[dvb.at[j % 6, 0]],
                             sem_a.at[j % 2], add=True)

        def _wait_scatter(j):
            pltpu.make_async_copy(rows.at[j % 3], acc.at[dvb.at[j % 6, 0]],
                                  sem_a.at[j % 2]).wait()

        _issue_swd(0)
        _issue_swd(1)
        _issue_swd(2)
        _wait_swd(0)
        _issue_xlate(0)
        _wait_xlate(0)
        _issue_rows(0)
        _wait_swd(1)
        _issue_xlate(1)

        def _chunk_body(j, _):
            @pl.when(j + 3 < n_chunks)
            def _():
                _issue_swd(j + 3)

            @pl.when(j + 2 < n_chunks)
            def _():
                _wait_swd(j + 2)
                _issue_xlate(j + 2)

            @pl.when(j + 1 < n_chunks)
            def _():
                _wait_xlate(j + 1)

                @pl.when(j >= 2)
                def _():
                    _wait_scatter(j - 2)
                _issue_rows(j + 1)

            _wait_rows(j)
            p = j % 3
            def _scale(e, _):
                wbc = plsc.load_gather(
                    wb.at[j % 4], [jnp.full((L,), e, jnp.int32)])
                for f in range(feat // L):
                    sl = pl.ds(f * L, L)
                    rows[p, e, sl] = rows[p, e, sl] * wbc
                return 0
            lax.fori_loop(0, chunk, _scale, 0)
            _issue_scatter(j)
            return 0

        lax.fori_loop(0, n_chunks, _chunk_body, 0)
        _wait_scatter(n_chunks - 2)
        _wait_scatter(n_chunks - 1)
        plsc.subcore_barrier()

        for k in range(rows_per_tile // chunk):
            off = sid * rows_per_tile + k * chunk
            pltpu.sync_copy(acc.at[pl.ds(off, chunk)],
                            out_hbm.at[cid, pl.ds(off, chunk)])

    return sc_kernel


def _tc_add(a, b):
    n, f = a.shape
    blk = 1024

    def body(a_ref, b_ref, o_ref):
        o_ref[...] = a_ref[...] + b_ref[...]

    return pl.pallas_call(
        body,
        grid=(n // blk,),
        in_specs=[pl.BlockSpec((blk, f), lambda i: (i, 0))] * 2,
        out_specs=pl.BlockSpec((blk, f), lambda i: (i, 0)),
        out_shape=jax.ShapeDtypeStruct((n, f), jnp.float32),
    )(a, b)


def kernel(runoff, pix_idxs, src_idxs, dest_idxs, weights):
    n_pix = pix_idxs.shape[0]
    e_total = src_idxs.shape[0]
    n_cats = 10000
    n_cats_pad = 10240
    feat = runoff.shape[1]
    chunk = 80

    pix32 = pix_idxs.astype(jnp.int32)
    src32 = src_idxs.astype(jnp.int32)
    per_w = e_total // NW
    dest32 = dest_idxs.astype(jnp.int32).reshape(NW, per_w // chunk, 1, chunk)

    sc = _sc_segment_matvec(n_cats_pad, n_pix, feat, e_total, chunk)
    partials = sc(runoff, pix32, src32, dest32, weights)
    out = _tc_add(partials[0], partials[1])
    return out[:n_cats][None]


# speedup vs baseline: 2.4102x; 1.0345x over previous
import functools

import jax
import jax.numpy as jnp
from jax import lax
from jax.experimental import pallas as pl
from jax.experimental.pallas import tpu as pltpu
from jax.experimental.pallas import tpu_sc as plsc

NC = 2
NS = 16
NW = NC * NS
L = 16


def _sc_segment_matvec(n_cats_pad, n_pix, feat, e_total, chunk):
    per_w = e_total // NW
    n_chunks = per_w // chunk
    rows_per_tile = n_cats_pad // NS
    assert n_cats_pad % (NS * 8) == 0 and per_w % chunk == 0
    assert rows_per_tile % chunk == 0 and chunk % L == 0

    mesh = plsc.VectorSubcoreMesh(core_axis_name="c", subcore_axis_name="s")

    @functools.partial(
        pl.kernel,
        out_type=jax.ShapeDtypeStruct((NC, n_cats_pad, feat), jnp.float32),
        mesh=mesh,
        scratch_types=[
            pltpu.VMEM_SHARED((n_pix,), jnp.int32),
            pltpu.VMEM_SHARED((n_cats_pad, feat), jnp.float32),
            pltpu.VMEM((6, 1, chunk), jnp.int32),
            pltpu.VMEM((4, chunk), jnp.int32),
            pltpu.VMEM((4, chunk), jnp.float32),
            pltpu.VMEM((3, chunk), jnp.int32),
            pltpu.VMEM((3, chunk, feat), jnp.float32),
            pltpu.SemaphoreType.DMA((4,)),
            pltpu.SemaphoreType.DMA((4,)),
            pltpu.SemaphoreType.DMA((6,)),
            pltpu.SemaphoreType.DMA((3,)),
            pltpu.SemaphoreType.DMA((3,)),
            pltpu.SemaphoreType.DMA((2,)),
        ],
        compiler_params=pltpu.CompilerParams(needs_layout_passes=False),
    )
    def sc_kernel(runoff_hbm, pix_hbm, src_hbm, dest_hbm, w_hbm, out_hbm,
                  pix_tab, acc, dvb, srcb, wb, gv, rows,
                  sem_s, sem_w, sem_d, sem_g, sem_r, sem_a):
        cid = lax.axis_index("c")
        sid = lax.axis_index("s")
        wid = sid * NC + cid

        def _zero_body(e, _):
            for f in range(feat // L):
                rows[0, e, pl.ds(f * L, L)] = jnp.zeros((L,), jnp.float32)
            return 0
        lax.fori_loop(0, chunk, _zero_body, 0)
        for k in range(rows_per_tile // chunk):
            pltpu.sync_copy(rows.at[0],
                            acc.at[pl.ds(sid * rows_per_tile + k * chunk,
                                         chunk)])

        @pl.when(sid == 0)
        def _():
            pltpu.sync_copy(pix_hbm, pix_tab)
        plsc.subcore_barrier()

        def _issue_swd(j):
            base = wid * per_w + j * chunk
            s = j % 4
            pltpu.async_copy(src_hbm.at[pl.ds(base, chunk)], srcb.at[s],
                             sem_s.at[s])
            pltpu.async_copy(w_hbm.at[pl.ds(base, chunk)], wb.at[s],
                             sem_w.at[s])
            pltpu.async_copy(dest_hbm.at[pl.ds(base, chunk)], dvb.at[j % 6, 0],
                             sem_d.at[j % 6])

        def _wait_swd(j):
            s = j % 4
            base = wid * per_w + j * chunk
            pltpu.make_async_copy(src_hbm.at[pl.ds(base, chunk)], srcb.at[s],
                                  sem_s.at[s]).wait()
            pltpu.make_async_copy(w_hbm.at[pl.ds(base, chunk)], wb.at[s],
                                  sem_w.at[s]).wait()
            pltpu.make_async_copy(dest_hbm.at[pl.ds(base, chunk)],
                                  dvb.at[j % 6, 0], sem_d.at[j % 6]).wait()

        def _issue_xlate(j):
            pltpu.async_copy(pix_tab.at[srcb.at[j % 4]], gv.at[j % 3],
                             sem_g.at[j % 3])

        def _wait_xlate(j):
            pltpu.make_async_copy(pix_tab.at[srcb.at[j % 4]], gv.at[j % 3],
                                  sem_g.at[j % 3]).wait()

        def _issue_rows(j):
            pltpu.async_copy(runoff_hbm.at[gv.at[j % 3]], rows.at[j % 3],
                             sem_r.at[j % 3])

        def _wait_rows(j):
            pltpu.make_async_copy(runoff_hbm.at[gv.at[j % 3]],
                                  rows.at[j % 3], sem_r.at[j % 3]).wait()

        def _issue_scatter(j):
            pltpu.async_copy(rows.at[j % 3], acc.at[dvb.at[j % 6, 0]],
                             sem_a.at[j % 2], add=True)

        def _wait_scatter(j):
            pltpu.make_async_copy(rows.at[j % 3], acc.at[dvb.at[j % 6, 0]],
                                  sem_a.at[j % 2]).wait()

        _issue_swd(0)
        _issue_swd(1)
        _issue_swd(2)
        _wait_swd(0)
        _issue_xlate(0)
        _wait_xlate(0)
        _issue_rows(0)
        _wait_swd(1)
        _issue_xlate(1)

        def _chunk_body(j, _):
            @pl.when(j + 3 < n_chunks)
            def _():
                _issue_swd(j + 3)

            @pl.when(j + 2 < n_chunks)
            def _():
                _wait_swd(j + 2)
                _issue_xlate(j + 2)

            @pl.when(j + 1 < n_chunks)
            def _():
                _wait_xlate(j + 1)

                @pl.when(j >= 2)
                def _():
                    _wait_scatter(j - 2)
                _issue_rows(j + 1)

            _wait_rows(j)
            p = j % 3
            def _scale(e, _):
                wbc = plsc.load_gather(
                    wb.at[j % 4], [jnp.full((L,), e, jnp.int32)])
                for f in range(feat // L):
                    sl = pl.ds(f * L, L)
                    rows[p, e, sl] = rows[p, e, sl] * wbc
                return 0
            lax.fori_loop(0, chunk, _scale, 0)
            _issue_scatter(j)
            return 0

        lax.fori_loop(0, n_chunks, _chunk_body, 0)
        _wait_scatter(n_chunks - 2)
        _wait_scatter(n_chunks - 1)
        plsc.subcore_barrier()

        for k in range(rows_per_tile // chunk):
            off = sid * rows_per_tile + k * chunk
            pltpu.sync_copy(acc.at[pl.ds(off, chunk)],
                            out_hbm.at[cid, pl.ds(off, chunk)])

    return sc_kernel


def _tc_add(a, b):
    n, f = a.shape
    blk = 1024

    def body(a_ref, b_ref, o_ref):
        o_ref[...] = a_ref[...] + b_ref[...]

    return pl.pallas_call(
        body,
        grid=(n // blk,),
        in_specs=[pl.BlockSpec((blk, f), lambda i: (i, 0))] * 2,
        out_specs=pl.BlockSpec((blk, f), lambda i: (i, 0)),
        out_shape=jax.ShapeDtypeStruct((n, f), jnp.float32),
    )(a, b)


def kernel(runoff, pix_idxs, src_idxs, dest_idxs, weights):
    n_pix = pix_idxs.shape[0]
    e_total = src_idxs.shape[0]
    n_cats = 10000
    n_cats_pad = 10240
    feat = runoff.shape[1]
    chunk = 80

    pix32 = pix_idxs.astype(jnp.int32)
    src32 = src_idxs.astype(jnp.int32)
    per_w = e_total // NW
    dest32 = dest_idxs.astype(jnp.int32)

    sc = _sc_segment_matvec(n_cats_pad, n_pix, feat, e_total, chunk)
    partials = sc(runoff, pix32, src32, dest32, weights)
    out = _tc_add(partials[0], partials[1])
    return out[:n_cats][None]
